# baseline (speedup 1.0000x reference)
import functools

import jax
import jax.numpy as jnp
from jax import lax
from jax.experimental import pallas as pl
from jax.experimental.pallas import tpu as pltpu
from jax.experimental.pallas import tpu_sc as plsc

NC = 2
NS = 16
NW = NC * NS

D = 64
BB = 128
NBUF = 2


def _gather_body(table_hbm, idxt_hbm, out5_hbm, idx_v, rows_v, lblk_v, *gsems,
                 n_blocks):
    wid = lax.axis_index("s") * NC + lax.axis_index("c")
    blocks_per_w = n_blocks // NW
    b0 = wid * blocks_per_w
    n_idx = blocks_per_w * BB

    pltpu.sync_copy(idxt_hbm.at[pl.ds(b0 * BB, n_idx)], idx_v)

    rvec = [lax.iota(jnp.int32, 16) + 16 * k for k in range(8)]

    def gather(j, b):
        pltpu.async_copy(
            table_hbm.at[idx_v.at[pl.ds(j * BB, BB)]], rows_v.at[b], gsems[b])

    def gather_wait(j, b):
        pltpu.make_async_copy(
            table_hbm.at[idx_v.at[pl.ds(j * BB, BB)]], rows_v.at[b], gsems[b]).wait()

    def transpose_block(b):
        for f in range(D):
            fvec = jnp.full((16,), f, jnp.int32)
            for k in range(8):
                col = plsc.load_gather(rows_v.at[b], [rvec[k], fvec])
                lblk_v[b, f // 8, f % 8, pl.ds(16 * k, 16)] = col

    def store_block(j, b):
        beta = b0 + j
        s = beta // 128
        bt = lax.rem(beta, 128)
        for ft in range(8):
            pltpu.sync_copy(lblk_v.at[b, ft], out5_hbm.at[s, ft, bt])

    for b in range(NBUF):
        gather(b, b)

    def group(g, _):
        for b in range(NBUF):
            j = g * NBUF + b
            gather_wait(j, b)
            transpose_block(b)
            gather(j + NBUF, b)
            store_block(j, b)
        return 0

    ngroups = blocks_per_w // NBUF
    lax.fori_loop(0, ngroups - 1, group, 0)

    for b in range(NBUF):
        j = (ngroups - 1) * NBUF + b
        gather_wait(j, b)
        transpose_block(b)
        store_block(j, b)


def kernel(input, weight):
    NB, S = input.shape
    B = NB * S
    n_blocks = B // BB
    idx_t = input.T.reshape(B)

    mesh = plsc.VectorSubcoreMesh(core_axis_name="c", subcore_axis_name="s")
    k = functools.partial(
        pl.kernel,
        out_type=jax.ShapeDtypeStruct((S, D // 8, NB // 128, 8, 128),
                                      jnp.float32),
        mesh=mesh,
        scratch_types=[
            pltpu.VMEM((B // NW,), jnp.int32),
            pltpu.VMEM((NBUF, BB, D), jnp.float32),
            pltpu.VMEM((NBUF, D // 8, 8, 128), jnp.float32),
        ] + [pltpu.SemaphoreType.DMA] * NBUF,
        compiler_params=pltpu.CompilerParams(use_tc_tiling_on_sc=False,
                                             needs_layout_passes=False),
    )(functools.partial(_gather_body, n_blocks=n_blocks))
    out5 = k(weight, idx_t)
    return out5.transpose(2, 4, 0, 1, 3).reshape(NB, S, D)


# speedup vs baseline: 1.1711x; 1.1711x over previous
import functools

import jax
import jax.numpy as jnp
from jax import lax
from jax.experimental import pallas as pl
from jax.experimental.pallas import tpu as pltpu
from jax.experimental.pallas import tpu_sc as plsc

NC = 2
NS = 16
NW = NC * NS

D = 64
BB = 128
NBUF = 4


def _gather_body(table_hbm, idxt_hbm, out5_hbm, idx_v, rows_v, lblk_v,
                 gsems, ssems, *, n_blocks):
    wid = lax.axis_index("s") * NC + lax.axis_index("c")
    blocks_per_w = n_blocks // NW
    b0 = wid * blocks_per_w
    n_idx = blocks_per_w * BB

    pltpu.sync_copy(idxt_hbm.at[pl.ds(b0 * BB, n_idx)], idx_v)

    rvec = [lax.iota(jnp.int32, 16) + 16 * k for k in range(8)]

    def gather(j, b):
        pltpu.async_copy(
            table_hbm.at[idx_v.at[pl.ds(j * BB, BB)]], rows_v.at[b], gsems.at[b])

    def gather_wait(j, b):
        pltpu.make_async_copy(
            table_hbm.at[idx_v.at[pl.ds(j * BB, BB)]], rows_v.at[b], gsems.at[b]).wait()

    def transpose_block(b):
        for f in range(D):
            fvec = jnp.full((16,), f, jnp.int32)
            for k in range(8):
                col = plsc.load_gather(rows_v.at[b], [rvec[k], fvec])
                lblk_v[b, f // 8, f % 8, pl.ds(16 * k, 16)] = col

    def store_block(j, b):
        beta = b0 + j
        s = beta // 128
        bt = lax.rem(beta, 128)
        for ft in range(8):
            pltpu.async_copy(lblk_v.at[b, ft], out5_hbm.at[s, ft, bt], ssems.at[b])

    def store_wait(j, b):
        beta = b0 + j
        s = beta // 128
        bt = lax.rem(beta, 128)
        for ft in range(8):
            pltpu.make_async_copy(
                lblk_v.at[b, ft], out5_hbm.at[s, ft, bt], ssems.at[b]).wait()

    for b in range(NBUF):
        gather(b, b)

    n = blocks_per_w

    def body(j, _):
        b = lax.rem(j, NBUF)
        gather_wait(j, b)

        @pl.when(j >= NBUF)
        def _():
            store_wait(j - NBUF, b)

        transpose_block(b)

        @pl.when(j < n - NBUF)
        def _():
            gather(j + NBUF, b)

        store_block(j, b)
        return 0

    lax.fori_loop(0, n, body, 0)

    def drain(j, _):
        store_wait(j, lax.rem(j, NBUF))
        return 0

    lax.fori_loop(n - NBUF, n, drain, 0)


def kernel(input, weight):
    NB, S = input.shape
    B = NB * S
    n_blocks = B // BB
    idx_t = input.T.reshape(B)

    mesh = plsc.VectorSubcoreMesh(core_axis_name="c", subcore_axis_name="s")
    k = functools.partial(
        pl.kernel,
        out_type=jax.ShapeDtypeStruct((S, D // 8, NB // 128, 8, 128),
                                      jnp.float32),
        mesh=mesh,
        scratch_types=[
            pltpu.VMEM((B // NW,), jnp.int32),
            pltpu.VMEM((NBUF, BB, D), jnp.float32),
            pltpu.VMEM((NBUF, D // 8, 8, 128), jnp.float32),
            pltpu.SemaphoreType.DMA((NBUF,)),
            pltpu.SemaphoreType.DMA((NBUF,)),
        ],
        compiler_params=pltpu.CompilerParams(use_tc_tiling_on_sc=False,
                                             needs_layout_passes=False,
                                             disable_bounds_checks=True),
    )(functools.partial(_gather_body, n_blocks=n_blocks))
    out5 = k(weight, idx_t)
    return out5.transpose(2, 4, 0, 1, 3).reshape(NB, S, D)


# speedup vs baseline: 1.7097x; 1.4600x over previous
import functools

import jax
import jax.numpy as jnp
from jax import lax
from jax.experimental import pallas as pl
from jax.experimental.pallas import tpu as pltpu
from jax.experimental.pallas import tpu_sc as plsc

NC = 2
NS = 16
NW = NC * NS

D = 64
BB = 128
NBUF = 4


def _gather_body(table_hbm, idxt_hbm, out5_hbm, idx_v, rows_v, lblk_v,
                 gsems, ssems, *, n_blocks):
    wid = lax.axis_index("s") * NC + lax.axis_index("c")
    blocks_per_w = n_blocks // NW
    b0 = wid * blocks_per_w
    n_idx = blocks_per_w * BB

    pltpu.sync_copy(idxt_hbm.at[pl.ds(b0 * BB, n_idx)], idx_v)

    rvec = [lax.iota(jnp.int32, 16) + 16 * k for k in range(8)]

    def gather(j, b):
        pltpu.async_copy(
            table_hbm.at[idx_v.at[pl.ds(j * BB, BB)]], rows_v.at[b], gsems.at[b])

    def gather_wait(j, b):
        pltpu.make_async_copy(
            table_hbm.at[idx_v.at[pl.ds(j * BB, BB)]], rows_v.at[b], gsems.at[b]).wait()

    def transpose_block(b):
        @plsc.parallel_loop(0, D, step=1)
        def _(f):
            fvec = jnp.full((16,), f, jnp.int32)
            for k in range(8):
                col = plsc.load_gather(rows_v.at[b], [rvec[k], fvec])
                lblk_v[b, f // 8, lax.rem(f, 8), pl.ds(16 * k, 16)] = col

    def store_block(j, b):
        beta = b0 + j
        s = beta // 128
        bt = lax.rem(beta, 128)
        for ft in range(8):
            pltpu.async_copy(lblk_v.at[b, ft], out5_hbm.at[s, ft, bt], ssems.at[b])

    def store_wait(j, b):
        beta = b0 + j
        s = beta // 128
        bt = lax.rem(beta, 128)
        for ft in range(8):
            pltpu.make_async_copy(
                lblk_v.at[b, ft], out5_hbm.at[s, ft, bt], ssems.at[b]).wait()

    for b in range(NBUF):
        gather(b, b)

    n = blocks_per_w

    def body(j, _):
        b = lax.rem(j, NBUF)
        gather_wait(j, b)

        @pl.when(j >= NBUF)
        def _():
            store_wait(j - NBUF, b)

        transpose_block(b)

        @pl.when(j < n - NBUF)
        def _():
            gather(j + NBUF, b)

        store_block(j, b)
        return 0

    lax.fori_loop(0, n, body, 0)

    def drain(j, _):
        store_wait(j, lax.rem(j, NBUF))
        return 0

    lax.fori_loop(n - NBUF, n, drain, 0)


def kernel(input, weight):
    NB, S = input.shape
    B = NB * S
    n_blocks = B // BB
    idx_t = input.T.reshape(B)

    mesh = plsc.VectorSubcoreMesh(core_axis_name="c", subcore_axis_name="s")
    k = functools.partial(
        pl.kernel,
        out_type=jax.ShapeDtypeStruct((S, D // 8, NB // 128, 8, 128),
                                      jnp.float32),
        mesh=mesh,
        scratch_types=[
            pltpu.VMEM((B // NW,), jnp.int32),
            pltpu.VMEM((NBUF, BB, D), jnp.float32),
            pltpu.VMEM((NBUF, D // 8, 8, 128), jnp.float32),
            pltpu.SemaphoreType.DMA((NBUF,)),
            pltpu.SemaphoreType.DMA((NBUF,)),
        ],
        compiler_params=pltpu.CompilerParams(use_tc_tiling_on_sc=False,
                                             needs_layout_passes=False,
                                             disable_bounds_checks=True),
    )(functools.partial(_gather_body, n_blocks=n_blocks))
    out5 = k(weight, idx_t)
    return out5.transpose(2, 4, 0, 1, 3).reshape(NB, S, D)


# speedup vs baseline: 1.7961x; 1.0505x over previous
import functools

import jax
import jax.numpy as jnp
from jax import lax
from jax.experimental import pallas as pl
from jax.experimental.pallas import tpu as pltpu
from jax.experimental.pallas import tpu_sc as plsc

NC = 2
NS = 16
NW = NC * NS

D = 64
BB = 128
NBUF = 4


def _gather_body(table_hbm, idxt_hbm, out5_hbm, idx_v, rows_v, lblk_v,
                 gsems, ssems, *, n_blocks):
    wid = lax.axis_index("s") * NC + lax.axis_index("c")
    blocks_per_w = n_blocks // NW
    b0 = wid * blocks_per_w
    n_idx = blocks_per_w * BB

    pltpu.sync_copy(idxt_hbm.at[pl.ds(b0 * BB, n_idx)], idx_v)

    rvec = [lax.iota(jnp.int32, 16) + 16 * k for k in range(8)]

    def gather(j, b):
        pltpu.async_copy(
            table_hbm.at[idx_v.at[pl.ds(j * BB, BB)]], rows_v.at[b], gsems.at[b])

    def gather_wait(j, b):
        pltpu.make_async_copy(
            table_hbm.at[idx_v.at[pl.ds(j * BB, BB)]], rows_v.at[b], gsems.at[b]).wait()

    def transpose_block(b):
        @plsc.parallel_loop(0, D, step=1)
        def _(f):
            fvec = jnp.full((16,), f, jnp.int32)
            for k in range(8):
                col = plsc.load_gather(rows_v.at[b], [rvec[k], fvec])
                lblk_v[b, f // 8, lax.rem(f, 8), pl.ds(16 * k, 16)] = col

    def store_block(j, b):
        beta = b0 + j
        s = beta // 128
        bt = lax.rem(beta, 128)
        for ft in range(8):
            pltpu.async_copy(lblk_v.at[b, ft], out5_hbm.at[s, ft, bt], ssems.at[b])

    def store_wait(j, b):
        beta = b0 + j
        s = beta // 128
        bt = lax.rem(beta, 128)
        for ft in range(8):
            pltpu.make_async_copy(
                lblk_v.at[b, ft], out5_hbm.at[s, ft, bt], ssems.at[b]).wait()

    for b in range(NBUF):
        gather(b, b)

    n = blocks_per_w

    def body(j, _):
        b = lax.rem(j, NBUF)
        gather_wait(j, b)

        @pl.when(j >= NBUF)
        def _():
            store_wait(j - NBUF, b)

        transpose_block(b)

        @pl.when(j < n - NBUF)
        def _():
            gather(j + NBUF, b)

        store_block(j, b)
        return 0

    lax.fori_loop(0, n, body, 0)

    def drain(j, _):
        store_wait(j, lax.rem(j, NBUF))
        return 0

    lax.fori_loop(n - NBUF, n, drain, 0)


def kernel(input, weight):
    NB, S = input.shape
    B = NB * S
    n_blocks = B // BB
    idx_t = input.T.reshape(B)
    weight = jnp.pad(weight, ((0, 0), (0, 128 - D)))

    mesh = plsc.VectorSubcoreMesh(core_axis_name="c", subcore_axis_name="s")
    k = functools.partial(
        pl.kernel,
        out_type=jax.ShapeDtypeStruct((S, D // 8, NB // 128, 8, 128),
                                      jnp.float32),
        mesh=mesh,
        scratch_types=[
            pltpu.VMEM((B // NW,), jnp.int32),
            pltpu.VMEM((NBUF, BB, 128), jnp.float32),
            pltpu.VMEM((NBUF, D // 8, 8, 128), jnp.float32),
            pltpu.SemaphoreType.DMA((NBUF,)),
            pltpu.SemaphoreType.DMA((NBUF,)),
        ],
        compiler_params=pltpu.CompilerParams(use_tc_tiling_on_sc=False,
                                             needs_layout_passes=False,
                                             disable_bounds_checks=True),
    )(functools.partial(_gather_body, n_blocks=n_blocks))
    out5 = k(weight, idx_t)
    return out5.transpose(2, 4, 0, 1, 3).reshape(NB, S, D)


# speedup vs baseline: 2.8259x; 1.5733x over previous
import functools

import jax
import jax.numpy as jnp
from jax import lax
from jax.experimental import pallas as pl
from jax.experimental.pallas import tpu as pltpu
from jax.experimental.pallas import tpu_sc as plsc

NC = 2
NS = 16
NW = NC * NS

D = 64
BB = 128
NBUF = 4


def _gather_body(table_hbm, idxt_hbm, out5_hbm, idx_v, rows_v, lblk_v,
                 gsems, ssems, *, n_blocks):
    wid = lax.axis_index("s") * NC + lax.axis_index("c")
    blocks_per_w = n_blocks // NW
    b0 = wid * blocks_per_w
    n_idx = blocks_per_w * BB

    pltpu.sync_copy(idxt_hbm.at[pl.ds(b0 * BB, n_idx)], idx_v)

    iot = lax.iota(jnp.int32, 16)
    ftv = [lax.shift_right_logical(iot + 16 * k, 3) for k in range(4)]
    fiv = [(iot + 16 * k) & 7 for k in range(4)]

    def gather(j, b):
        pltpu.async_copy(
            table_hbm.at[idx_v.at[pl.ds(j * BB, BB)]], rows_v.at[b], gsems.at[b])

    def gather_wait(j, b):
        pltpu.make_async_copy(
            table_hbm.at[idx_v.at[pl.ds(j * BB, BB)]], rows_v.at[b], gsems.at[b]).wait()

    def transpose_block(b):
        @plsc.parallel_loop(0, BB, step=1)
        def _(bi):
            bivec = jnp.full((16,), bi, jnp.int32)
            for k in range(D // 16):
                v = rows_v[b, bi, pl.ds(16 * k, 16)]
                plsc.store_scatter(lblk_v.at[b], [ftv[k], fiv[k], bivec], v)

    def store_block(j, b):
        beta = b0 + j
        s = beta // 128
        bt = lax.rem(beta, 128)
        for ft in range(8):
            pltpu.async_copy(lblk_v.at[b, ft, :, pl.ds(0, 128)],
                             out5_hbm.at[s, ft, bt], ssems.at[b])

    def store_wait(j, b):
        beta = b0 + j
        s = beta // 128
        bt = lax.rem(beta, 128)
        for ft in range(8):
            pltpu.make_async_copy(
                lblk_v.at[b, ft, :, pl.ds(0, 128)],
                out5_hbm.at[s, ft, bt], ssems.at[b]).wait()

    for b in range(NBUF):
        gather(b, b)

    n = blocks_per_w

    def body(j, _):
        b = lax.rem(j, NBUF)
        gather_wait(j, b)

        @pl.when(j >= NBUF)
        def _():
            store_wait(j - NBUF, b)

        transpose_block(b)

        @pl.when(j < n - NBUF)
        def _():
            gather(j + NBUF, b)

        store_block(j, b)
        return 0

    lax.fori_loop(0, n, body, 0)

    def drain(j, _):
        store_wait(j, lax.rem(j, NBUF))
        return 0

    lax.fori_loop(n - NBUF, n, drain, 0)


def kernel(input, weight):
    NB, S = input.shape
    B = NB * S
    n_blocks = B // BB
    idx_t = input.T.reshape(B)
    weight = jnp.pad(weight, ((0, 0), (0, 128 - D)))

    mesh = plsc.VectorSubcoreMesh(core_axis_name="c", subcore_axis_name="s")
    k = functools.partial(
        pl.kernel,
        out_type=jax.ShapeDtypeStruct((S, D // 8, NB // 128, 8, 128),
                                      jnp.float32),
        mesh=mesh,
        scratch_types=[
            pltpu.VMEM((B // NW,), jnp.int32),
            pltpu.VMEM((NBUF, BB, 128), jnp.float32),
            pltpu.VMEM((NBUF, D // 8, 8, 132), jnp.float32),
            pltpu.SemaphoreType.DMA((NBUF,)),
            pltpu.SemaphoreType.DMA((NBUF,)),
        ],
        compiler_params=pltpu.CompilerParams(use_tc_tiling_on_sc=False,
                                             needs_layout_passes=False,
                                             disable_bounds_checks=True),
    )(functools.partial(_gather_body, n_blocks=n_blocks))
    out5 = k(weight, idx_t)
    return out5.transpose(2, 4, 0, 1, 3).reshape(NB, S, D)
